# baseline (speedup 1.0000x reference)
import functools

import jax
import jax.numpy as jnp
from jax import lax
from jax.experimental import pallas as pl
from jax.experimental.pallas import tpu as pltpu
from jax.experimental.pallas import tpu_sc as plsc

NUM_CHEM = 6000
NUM_DIS = 4000
HIDDEN = 128
E_POS = 160000
BATCH = 16384

NC_P = 6144
ND_P = 4096
NN_P = NC_P + ND_P

NT = 16
CHUNK = 128
E_TILE = 10240
E_PAD = NT * E_TILE
NCHUNK = E_TILE // CHUNK

_mesh = plsc.VectorSubcoreMesh(core_axis_name="c", subcore_axis_name="s")


def _make_seg_sums(with_cnt):
    out_type = [
        jax.ShapeDtypeStruct((NN_P, HIDDEN), jnp.float32),
    ]
    scratch = [
        pltpu.VMEM((NCHUNK, CHUNK), jnp.int32),
        pltpu.VMEM((NCHUNK, CHUNK), jnp.int32),
        pltpu.VMEM((2, CHUNK, HIDDEN), jnp.float32),
        pltpu.VMEM((32, HIDDEN), jnp.float32),
        pltpu.VMEM((CHUNK,), jnp.float32),
        pltpu.VMEM_SHARED((NC_P, HIDDEN), jnp.float32),
        pltpu.VMEM_SHARED((NC_P,), jnp.float32),
    ] + [pltpu.SemaphoreType.DMA] * 5
    if with_cnt:
        out_type = out_type + [
            jax.ShapeDtypeStruct((NN_P,), jnp.float32),
        ]
    @functools.partial(pl.kernel, mesh=_mesh, out_type=out_type,
                       scratch_types=scratch)
    def _seg(x_hbm, tc_hbm, td_hbm, tdo_hbm, sum_hbm, *rest):
        if with_cnt:
            cnt_hbm = rest[0]
            rest = rest[1:]
        (sidx, didx, rows, zbuf, ones, acc, cacc, *sems) = rest
        gsem = sems[0:2]
        ssem = sems[2:4]
        csem = sems[4]
        cid = lax.axis_index("c")
        sid = lax.axis_index("s")

        for i in range(32):
            for j in range(HIDDEN // 16):
                zbuf[i, pl.ds(j * 16, 16)] = jnp.zeros((16,), jnp.float32)
        for j in range(CHUNK // 16):
            ones[pl.ds(j * 16, 16)] = jnp.ones((16,), jnp.float32)

        r0 = ND_P // NT
        r1 = NC_P // NT
        for b in range(r1 // 32):
            pltpu.sync_copy(zbuf, acc.at[pl.ds(sid * r1 + b * 32, 32)])
        if with_cnt:
            for b in range(r1 // CHUNK):
                pltpu.sync_copy(zbuf.at[0],
                                cacc.at[pl.ds(sid * r1 + b * CHUNK, CHUNK)])
        plsc.subcore_barrier()

        def do_rel(src3_hbm, x_hbm, dst3_hbm, acc):
            pltpu.sync_copy(src3_hbm.at[sid], sidx)
            pltpu.sync_copy(dst3_hbm.at[sid], didx)

            def gather(k, b):
                return pltpu.make_async_copy(
                    x_hbm.at[sidx.at[k]], rows.at[b], gsem[b])

            def scatter_wait(k, b):
                pltpu.make_async_copy(
                    rows.at[b], acc.at[didx.at[k]], ssem[b]).wait()

            def step(k, b):
                gather(k, b).wait()
                pltpu.async_copy(rows.at[b], acc.at[didx.at[k]], ssem[b],
                                 add=True)
                if with_cnt:
                    pltpu.async_copy(ones, cacc.at[didx.at[k]], csem,
                                     add=True)

            gather(0, 0).start()

            def body(j, carry):
                for b in range(2):
                    k = 2 * j + b
                    step(k, b)

                    @pl.when(k + 1 < NCHUNK)
                    def _():
                        @pl.when(k >= 1)
                        def _():
                            scatter_wait(k - 1, 1 - b)
                        gather(k + 1, 1 - b).start()
                return carry
            lax.fori_loop(0, NCHUNK // 2, body, 0)
            scatter_wait(NCHUNK - 2, 0)
            scatter_wait(NCHUNK - 1, 1)
            if with_cnt:
                def cbody(k, carry):
                    pltpu.make_async_copy(
                        ones, cacc.at[didx.at[0]], csem).wait()
                    return carry
                lax.fori_loop(0, NCHUNK, cbody, 0)

        @pl.when(cid == 0)
        def _():
            do_rel(tc_hbm, x_hbm, td_hbm, acc)

        @pl.when(cid == 1)
        def _():
            do_rel(tdo_hbm, x_hbm, tc_hbm, acc)

        plsc.subcore_barrier()

        @pl.when(cid == 0)
        def _():
            pltpu.sync_copy(acc.at[pl.ds(sid * r0, r0)],
                            sum_hbm.at[pl.ds(NC_P + sid * r0, r0)])
            if with_cnt:
                @pl.when(sid == 0)
                def _():
                    pltpu.sync_copy(cacc.at[pl.ds(0, ND_P)],
                                    cnt_hbm.at[pl.ds(NC_P, ND_P)])

        @pl.when(cid == 1)
        def _():
            pltpu.sync_copy(acc.at[pl.ds(sid * r1, r1)],
                            sum_hbm.at[pl.ds(sid * r1, r1)])
            if with_cnt:
                @pl.when(sid == 0)
                def _():
                    pltpu.sync_copy(cacc, cnt_hbm.at[pl.ds(0, NC_P)])

    return _seg


_seg_sums_l1 = _make_seg_sums(True)
_seg_sums_l2 = _make_seg_sums(False)


B_TILE = BATCH // 32


@functools.partial(
    pl.kernel,
    mesh=_mesh,
    out_type=[
        jax.ShapeDtypeStruct((BATCH, HIDDEN), jnp.float32),
        jax.ShapeDtypeStruct((BATCH, HIDDEN), jnp.float32),
    ],
    scratch_types=[
        pltpu.VMEM((CHUNK,), jnp.int32),
        pltpu.VMEM((CHUNK, HIDDEN), jnp.float32),
        pltpu.SemaphoreType.DMA,
    ],
)
def _pair_gather(x_hbm, cid_hbm, did_hbm, cout_hbm, dout_hbm,
                 idxv, rows, sem):
    cid = lax.axis_index("c")
    sid = lax.axis_index("s")
    wid = sid * 2 + cid
    base = wid * B_TILE

    def do_tab(ids_hbm, out_hbm):
        def body(k, carry):
            off = base + k * CHUNK
            pltpu.sync_copy(ids_hbm.at[pl.ds(off, CHUNK)], idxv)
            pltpu.async_copy(x_hbm.at[idxv], rows, sem).wait()
            pltpu.sync_copy(rows, out_hbm.at[pl.ds(off, CHUNK)])
            return carry
        lax.fori_loop(0, B_TILE // CHUNK, body, 0)

    do_tab(cid_hbm, cout_hbm)
    do_tab(did_hbm, dout_hbm)


def _combine_body(x_ref, s_ref, cnt_ref, wroot_ref, wrel_ref, b_ref, o_ref,
                  *, relu):
    scale = 1.0 / jnp.maximum(cnt_ref[...], 1.0)
    acc = jnp.dot(x_ref[...], wroot_ref[...],
                  preferred_element_type=jnp.float32)
    acc = acc + b_ref[...]
    acc = acc + jnp.dot(s_ref[...] * scale, wrel_ref[0],
                        preferred_element_type=jnp.float32)
    if relu:
        acc = jnp.maximum(acc, 0.0)
    o_ref[...] = acc


_CBLK = 512
_NCB = NC_P // _CBLK


def _combine(x, s, cnt, wroot, wrel_l, b, relu):
    kern = functools.partial(_combine_body, relu=relu)
    return pl.pallas_call(
        kern,
        grid=(NN_P // _CBLK,),
        in_specs=[
            pl.BlockSpec((_CBLK, HIDDEN), lambda i: (i, 0)),
            pl.BlockSpec((_CBLK, HIDDEN), lambda i: (i, 0)),
            pl.BlockSpec((_CBLK, 1), lambda i: (i, 0)),
            pl.BlockSpec((HIDDEN, HIDDEN), lambda i: (0, 0)),
            pl.BlockSpec((1, HIDDEN, HIDDEN),
                         lambda i: (jnp.where(i < _NCB, 1, 0), 0, 0)),
            pl.BlockSpec((1, HIDDEN), lambda i: (0, 0)),
        ],
        out_specs=pl.BlockSpec((_CBLK, HIDDEN), lambda i: (i, 0)),
        out_shape=jax.ShapeDtypeStruct((NN_P, HIDDEN), jnp.float32),
    )(x, s, cnt, wroot, wrel_l, b)


def _score_body(c_ref, d_ref, w_ref, o_ref):
    cw = jnp.dot(c_ref[...], w_ref[...], preferred_element_type=jnp.float32)
    o_ref[...] = jnp.sum(cw * d_ref[...], axis=1, keepdims=True)


def _score(c, d, w):
    blk = 512
    return pl.pallas_call(
        _score_body,
        grid=(BATCH // blk,),
        in_specs=[
            pl.BlockSpec((blk, HIDDEN), lambda i: (i, 0)),
            pl.BlockSpec((blk, HIDDEN), lambda i: (i, 0)),
            pl.BlockSpec((HIDDEN, HIDDEN), lambda i: (0, 0)),
        ],
        out_specs=pl.BlockSpec((blk, 1), lambda i: (i, 0)),
        out_shape=jax.ShapeDtypeStruct((BATCH, 1), jnp.float32),
    )(c, d, w)


def kernel(chem_ids, dis_ids, train_chem, train_dis, node_emb, w_rel, w_root,
           bias, W):
    f32 = jnp.float32
    pad_e = E_PAD - E_POS
    tc_i = train_chem.astype(jnp.int32)
    td_i = train_dis.astype(jnp.int32)
    tc_pad = jnp.concatenate(
        [tc_i, jnp.full((pad_e,), NUM_CHEM, jnp.int32)]
    ).reshape(NT, NCHUNK, CHUNK)
    td_pad = jnp.concatenate(
        [td_i, jnp.full((pad_e,), NUM_DIS, jnp.int32)]
    ).reshape(NT, NCHUNK, CHUNK)
    td_ofs = td_pad + NC_P

    x = (jnp.zeros((NN_P, HIDDEN), f32)
         .at[:NUM_CHEM].set(node_emb[:NUM_CHEM])
         .at[NC_P:NC_P + NUM_DIS].set(node_emb[NUM_CHEM:]))

    cnt_keep = None
    for l in range(2):
        if l == 0:
            s, cnt = _seg_sums_l1(x, tc_pad, td_pad, td_ofs)
            cnt_keep = cnt.reshape(NN_P, 1)
        else:
            s = _seg_sums_l2(x, tc_pad, td_pad, td_ofs)
            if isinstance(s, (list, tuple)):
                s = s[0]
        x = _combine(x, s, cnt_keep, w_root[l], w_rel[l],
                     bias[l].reshape(1, HIDDEN), relu=(l == 0))

    c_rows, d_rows = _pair_gather(x, chem_ids.astype(jnp.int32),
                                  dis_ids.astype(jnp.int32) + NC_P)
    return _score(c_rows, d_rows, W)[:, 0]


# speedup vs baseline: 1.0499x; 1.0499x over previous
import functools

import jax
import jax.numpy as jnp
from jax import lax
from jax.experimental import pallas as pl
from jax.experimental.pallas import tpu as pltpu
from jax.experimental.pallas import tpu_sc as plsc

NUM_CHEM = 6000
NUM_DIS = 4000
HIDDEN = 128
E_POS = 160000
BATCH = 16384

NC_P = 6144
ND_P = 4096
NN_P = NC_P + ND_P

NT = 16
CHUNK = 128
E_TILE = 10240
E_PAD = NT * E_TILE
NCHUNK = E_TILE // CHUNK

_mesh = plsc.VectorSubcoreMesh(core_axis_name="c", subcore_axis_name="s")


def _make_seg_sums(with_cnt):
    out_type = [
        jax.ShapeDtypeStruct((NN_P, HIDDEN), jnp.float32),
    ]
    scratch = [
        pltpu.VMEM((8, CHUNK), jnp.int32),
        pltpu.VMEM((8, CHUNK), jnp.int32),
        pltpu.VMEM((4, CHUNK, HIDDEN), jnp.float32),
        pltpu.VMEM((32, HIDDEN), jnp.float32),
        pltpu.VMEM((CHUNK,), jnp.float32),
        pltpu.VMEM_SHARED((NC_P, HIDDEN), jnp.float32),
        pltpu.VMEM_SHARED((NC_P,), jnp.float32),
    ] + [pltpu.SemaphoreType.DMA] * 17
    if with_cnt:
        out_type = out_type + [
            jax.ShapeDtypeStruct((NN_P,), jnp.float32),
        ]
    @functools.partial(pl.kernel, mesh=_mesh, out_type=out_type,
                       scratch_types=scratch)
    def _seg(x_hbm, tc_hbm, td_hbm, tdo_hbm, sum_hbm, *rest):
        if with_cnt:
            cnt_hbm = rest[0]
            rest = rest[1:]
        (sidx, didx, rows, zbuf, ones, acc, cacc, *sems) = rest
        gsem, ssem, isem, csem = sems[0:4], sems[4:8], sems[8:16], sems[16]
        cid = lax.axis_index("c")
        sid = lax.axis_index("s")

        for i in range(32):
            for j in range(HIDDEN // 16):
                zbuf[i, pl.ds(j * 16, 16)] = jnp.zeros((16,), jnp.float32)
        for j in range(CHUNK // 16):
            ones[pl.ds(j * 16, 16)] = jnp.ones((16,), jnp.float32)

        r0 = ND_P // NT
        r1 = NC_P // NT
        for b in range(r1 // 32):
            pltpu.sync_copy(zbuf, acc.at[pl.ds(sid * r1 + b * 32, 32)])
        if with_cnt:
            for b in range(r1 // CHUNK):
                pltpu.sync_copy(zbuf.at[0],
                                cacc.at[pl.ds(sid * r1 + b * CHUNK, CHUNK)])
        plsc.subcore_barrier()

        def do_rel(src3_hbm, x_hbm, dst3_hbm, acc):
            def idx(k, i):
                return (pltpu.make_async_copy(
                            src3_hbm.at[sid].at[k], sidx.at[i], isem[i]),
                        pltpu.make_async_copy(
                            dst3_hbm.at[sid].at[k], didx.at[i], isem[i]))

            def gather(k, i, b):
                return pltpu.make_async_copy(
                    x_hbm.at[sidx.at[i]], rows.at[b], gsem[b])

            def scatter(k, i, b):
                return pltpu.make_async_copy(
                    rows.at[b], acc.at[didx.at[i]], ssem[b])

            for k in range(4):
                a, d = idx(k, k)
                a.start()
                d.start()
            for k in range(2):
                a, d = idx(k, k)
                a.wait()
                d.wait()
                gather(k, k, k).start()

            def body(j, carry):
                for b in range(8):
                    k = 8 * j + b
                    br = b % 4

                    @pl.when(k >= 2)
                    def _():
                        scatter(k - 2, (b + 6) % 8, (br + 2) % 4).wait()
                    gather(k, b, br).wait()
                    pltpu.async_copy(rows.at[br], acc.at[didx.at[b]],
                                     ssem[br], add=True)
                    if with_cnt:
                        pltpu.async_copy(ones, cacc.at[didx.at[b]], csem,
                                         add=True)

                    @pl.when(k + 2 < NCHUNK)
                    def _():
                        a, d = idx(k + 2, (b + 2) % 8)
                        a.wait()
                        d.wait()
                        gather(k + 2, (b + 2) % 8, (br + 2) % 4).start()

                    @pl.when(k + 4 < NCHUNK)
                    def _():
                        a, d = idx(k + 4, (b + 4) % 8)
                        a.start()
                        d.start()
                return carry
            lax.fori_loop(0, NCHUNK // 8, body, 0)

            scatter(NCHUNK - 2, 6, 2).wait()
            scatter(NCHUNK - 1, 7, 3).wait()
            if with_cnt:
                def cbody(k, carry):
                    pltpu.make_async_copy(
                        ones, cacc.at[didx.at[0]], csem).wait()
                    return carry
                lax.fori_loop(0, NCHUNK, cbody, 0)

        @pl.when(cid == 0)
        def _():
            do_rel(tc_hbm, x_hbm, td_hbm, acc)

        @pl.when(cid == 1)
        def _():
            do_rel(tdo_hbm, x_hbm, tc_hbm, acc)

        plsc.subcore_barrier()

        @pl.when(cid == 0)
        def _():
            pltpu.sync_copy(acc.at[pl.ds(sid * r0, r0)],
                            sum_hbm.at[pl.ds(NC_P + sid * r0, r0)])
            if with_cnt:
                @pl.when(sid == 0)
                def _():
                    pltpu.sync_copy(cacc.at[pl.ds(0, ND_P)],
                                    cnt_hbm.at[pl.ds(NC_P, ND_P)])

        @pl.when(cid == 1)
        def _():
            pltpu.sync_copy(acc.at[pl.ds(sid * r1, r1)],
                            sum_hbm.at[pl.ds(sid * r1, r1)])
            if with_cnt:
                @pl.when(sid == 0)
                def _():
                    pltpu.sync_copy(cacc, cnt_hbm.at[pl.ds(0, NC_P)])

    return _seg


_seg_sums_l1 = _make_seg_sums(True)
_seg_sums_l2 = _make_seg_sums(False)


B_TILE = BATCH // 32


@functools.partial(
    pl.kernel,
    mesh=_mesh,
    out_type=[
        jax.ShapeDtypeStruct((BATCH, HIDDEN), jnp.float32),
        jax.ShapeDtypeStruct((BATCH, HIDDEN), jnp.float32),
    ],
    scratch_types=[
        pltpu.VMEM((CHUNK,), jnp.int32),
        pltpu.VMEM((CHUNK, HIDDEN), jnp.float32),
        pltpu.SemaphoreType.DMA,
    ],
)
def _pair_gather(x_hbm, cid_hbm, did_hbm, cout_hbm, dout_hbm,
                 idxv, rows, sem):
    cid = lax.axis_index("c")
    sid = lax.axis_index("s")
    wid = sid * 2 + cid
    base = wid * B_TILE

    def do_tab(ids_hbm, out_hbm):
        def body(k, carry):
            off = base + k * CHUNK
            pltpu.sync_copy(ids_hbm.at[pl.ds(off, CHUNK)], idxv)
            pltpu.async_copy(x_hbm.at[idxv], rows, sem).wait()
            pltpu.sync_copy(rows, out_hbm.at[pl.ds(off, CHUNK)])
            return carry
        lax.fori_loop(0, B_TILE // CHUNK, body, 0)

    do_tab(cid_hbm, cout_hbm)
    do_tab(did_hbm, dout_hbm)


def _combine_body(x_ref, s_ref, cnt_ref, wroot_ref, wrel_ref, b_ref, o_ref,
                  *, relu):
    scale = 1.0 / jnp.maximum(cnt_ref[...], 1.0)
    acc = jnp.dot(x_ref[...], wroot_ref[...],
                  preferred_element_type=jnp.float32)
    acc = acc + b_ref[...]
    acc = acc + jnp.dot(s_ref[...] * scale, wrel_ref[0],
                        preferred_element_type=jnp.float32)
    if relu:
        acc = jnp.maximum(acc, 0.0)
    o_ref[...] = acc


_CBLK = 512
_NCB = NC_P // _CBLK


def _combine(x, s, cnt, wroot, wrel_l, b, relu):
    kern = functools.partial(_combine_body, relu=relu)
    return pl.pallas_call(
        kern,
        grid=(NN_P // _CBLK,),
        in_specs=[
            pl.BlockSpec((_CBLK, HIDDEN), lambda i: (i, 0)),
            pl.BlockSpec((_CBLK, HIDDEN), lambda i: (i, 0)),
            pl.BlockSpec((_CBLK, 1), lambda i: (i, 0)),
            pl.BlockSpec((HIDDEN, HIDDEN), lambda i: (0, 0)),
            pl.BlockSpec((1, HIDDEN, HIDDEN),
                         lambda i: (jnp.where(i < _NCB, 1, 0), 0, 0)),
            pl.BlockSpec((1, HIDDEN), lambda i: (0, 0)),
        ],
        out_specs=pl.BlockSpec((_CBLK, HIDDEN), lambda i: (i, 0)),
        out_shape=jax.ShapeDtypeStruct((NN_P, HIDDEN), jnp.float32),
    )(x, s, cnt, wroot, wrel_l, b)


def _score_body(c_ref, d_ref, w_ref, o_ref):
    cw = jnp.dot(c_ref[...], w_ref[...], preferred_element_type=jnp.float32)
    o_ref[...] = jnp.sum(cw * d_ref[...], axis=1, keepdims=True)


def _score(c, d, w):
    blk = 512
    return pl.pallas_call(
        _score_body,
        grid=(BATCH // blk,),
        in_specs=[
            pl.BlockSpec((blk, HIDDEN), lambda i: (i, 0)),
            pl.BlockSpec((blk, HIDDEN), lambda i: (i, 0)),
            pl.BlockSpec((HIDDEN, HIDDEN), lambda i: (0, 0)),
        ],
        out_specs=pl.BlockSpec((blk, 1), lambda i: (i, 0)),
        out_shape=jax.ShapeDtypeStruct((BATCH, 1), jnp.float32),
    )(c, d, w)


def kernel(chem_ids, dis_ids, train_chem, train_dis, node_emb, w_rel, w_root,
           bias, W):
    f32 = jnp.float32
    pad_e = E_PAD - E_POS
    tc_i = train_chem.astype(jnp.int32)
    td_i = train_dis.astype(jnp.int32)
    tc_pad = jnp.concatenate(
        [tc_i, jnp.full((pad_e,), NUM_CHEM, jnp.int32)]
    ).reshape(NT, NCHUNK, CHUNK)
    td_pad = jnp.concatenate(
        [td_i, jnp.full((pad_e,), NUM_DIS, jnp.int32)]
    ).reshape(NT, NCHUNK, CHUNK)
    td_ofs = td_pad + NC_P

    x = (jnp.zeros((NN_P, HIDDEN), f32)
         .at[:NUM_CHEM].set(node_emb[:NUM_CHEM])
         .at[NC_P:NC_P + NUM_DIS].set(node_emb[NUM_CHEM:]))

    cnt_keep = None
    for l in range(2):
        if l == 0:
            s, cnt = _seg_sums_l1(x, tc_pad, td_pad, td_ofs)
            cnt_keep = cnt.reshape(NN_P, 1)
        else:
            s = _seg_sums_l2(x, tc_pad, td_pad, td_ofs)
            if isinstance(s, (list, tuple)):
                s = s[0]
        x = _combine(x, s, cnt_keep, w_root[l], w_rel[l],
                     bias[l].reshape(1, HIDDEN), relu=(l == 0))

    c_rows, d_rows = _pair_gather(x, chem_ids.astype(jnp.int32),
                                  dis_ids.astype(jnp.int32) + NC_P)
    return _score(c_rows, d_rows, W)[:, 0]


# speedup vs baseline: 1.0553x; 1.0051x over previous
import functools

import jax
import jax.numpy as jnp
from jax import lax
from jax.experimental import pallas as pl
from jax.experimental.pallas import tpu as pltpu
from jax.experimental.pallas import tpu_sc as plsc

NUM_CHEM = 6000
NUM_DIS = 4000
HIDDEN = 128
E_POS = 160000
BATCH = 16384

NC_P = 6144
ND_P = 4096
NN_P = NC_P + ND_P

NT = 16
CHUNK = 128
E_TILE = 10240
E_PAD = NT * E_TILE
NCHUNK = E_TILE // CHUNK

_mesh = plsc.VectorSubcoreMesh(core_axis_name="c", subcore_axis_name="s")


def _make_seg_sums(with_cnt):
    out_type = [
        jax.ShapeDtypeStruct((NN_P, HIDDEN), jnp.float32),
    ]
    scratch = [
        pltpu.VMEM((8, CHUNK), jnp.int32),
        pltpu.VMEM((8, CHUNK), jnp.int32),
        pltpu.VMEM((4, CHUNK, HIDDEN), jnp.float32),
        pltpu.VMEM((32, HIDDEN), jnp.float32),
        pltpu.VMEM((CHUNK,), jnp.float32),
        pltpu.VMEM_SHARED((NC_P, HIDDEN), jnp.float32),
        pltpu.VMEM_SHARED((NC_P,), jnp.float32),
    ] + [pltpu.SemaphoreType.DMA] * 17
    if with_cnt:
        out_type = out_type + [
            jax.ShapeDtypeStruct((NN_P,), jnp.float32),
        ]
    @functools.partial(pl.kernel, mesh=_mesh, out_type=out_type,
                       scratch_types=scratch)
    def _seg(x_hbm, tc_hbm, td_hbm, tdo_hbm, sum_hbm, *rest):
        if with_cnt:
            cnt_hbm = rest[0]
            rest = rest[1:]
        (sidx, didx, rows, zbuf, ones, acc, cacc, *sems) = rest
        gsem, ssem, isem, csem = sems[0:4], sems[4:8], sems[8:16], sems[16]
        cid = lax.axis_index("c")
        sid = lax.axis_index("s")

        for i in range(32):
            for j in range(HIDDEN // 16):
                zbuf[i, pl.ds(j * 16, 16)] = jnp.zeros((16,), jnp.float32)
        for j in range(CHUNK // 16):
            ones[pl.ds(j * 16, 16)] = jnp.ones((16,), jnp.float32)

        r0 = ND_P // NT
        r1 = NC_P // NT
        for b in range(r1 // 32):
            pltpu.sync_copy(zbuf, acc.at[pl.ds(sid * r1 + b * 32, 32)])
        if with_cnt:
            for b in range(r1 // CHUNK):
                pltpu.sync_copy(zbuf.at[0],
                                cacc.at[pl.ds(sid * r1 + b * CHUNK, CHUNK)])
        plsc.subcore_barrier()

        def do_rel(src3_hbm, x_hbm, dst3_hbm, acc):
            def idx(k, i):
                return (pltpu.make_async_copy(
                            src3_hbm.at[sid].at[k], sidx.at[i], isem[i]),
                        pltpu.make_async_copy(
                            dst3_hbm.at[sid].at[k], didx.at[i], isem[i]))

            def gather(k, i, b):
                return pltpu.make_async_copy(
                    x_hbm.at[sidx.at[i]], rows.at[b], gsem[b])

            def scatter(k, i, b):
                return pltpu.make_async_copy(
                    rows.at[b], acc.at[didx.at[i]], ssem[b])

            for k in range(4):
                a, d = idx(k, k)
                a.start()
                d.start()
            for k in range(2):
                a, d = idx(k, k)
                a.wait()
                d.wait()
                gather(k, k, k).start()

            def body(j, carry):
                for b in range(8):
                    k = 8 * j + b
                    br = b % 4

                    @pl.when(k >= 2)
                    def _():
                        scatter(k - 2, (b + 6) % 8, (br + 2) % 4).wait()
                    gather(k, b, br).wait()
                    pltpu.async_copy(rows.at[br], acc.at[didx.at[b]],
                                     ssem[br], add=True)
                    if with_cnt:
                        pltpu.async_copy(ones, cacc.at[didx.at[b]], csem,
                                         add=True)

                    @pl.when(k + 2 < NCHUNK)
                    def _():
                        a, d = idx(k + 2, (b + 2) % 8)
                        a.wait()
                        d.wait()
                        gather(k + 2, (b + 2) % 8, (br + 2) % 4).start()

                    @pl.when(k + 4 < NCHUNK)
                    def _():
                        a, d = idx(k + 4, (b + 4) % 8)
                        a.start()
                        d.start()
                return carry
            lax.fori_loop(0, NCHUNK // 8, body, 0)

            scatter(NCHUNK - 2, 6, 2).wait()
            scatter(NCHUNK - 1, 7, 3).wait()
            if with_cnt:
                def cbody(k, carry):
                    pltpu.make_async_copy(
                        ones, cacc.at[didx.at[0]], csem).wait()
                    return carry
                lax.fori_loop(0, NCHUNK, cbody, 0)

        @pl.when(cid == 0)
        def _():
            do_rel(tc_hbm, x_hbm, td_hbm, acc)

        @pl.when(cid == 1)
        def _():
            do_rel(tdo_hbm, x_hbm, tc_hbm, acc)

        plsc.subcore_barrier()

        @pl.when(cid == 0)
        def _():
            pltpu.sync_copy(acc.at[pl.ds(sid * r0, r0)],
                            sum_hbm.at[pl.ds(NC_P + sid * r0, r0)])
            if with_cnt:
                @pl.when(sid == 0)
                def _():
                    pltpu.sync_copy(cacc.at[pl.ds(0, ND_P)],
                                    cnt_hbm.at[pl.ds(NC_P, ND_P)])

        @pl.when(cid == 1)
        def _():
            pltpu.sync_copy(acc.at[pl.ds(sid * r1, r1)],
                            sum_hbm.at[pl.ds(sid * r1, r1)])
            if with_cnt:
                @pl.when(sid == 0)
                def _():
                    pltpu.sync_copy(cacc, cnt_hbm.at[pl.ds(0, NC_P)])

    return _seg


_seg_sums_l1 = _make_seg_sums(True)
_seg_sums_l2 = _make_seg_sums(False)


B_TILE = BATCH // 32


_GCH = B_TILE // CHUNK


@functools.partial(
    pl.kernel,
    mesh=_mesh,
    out_type=[
        jax.ShapeDtypeStruct((BATCH, HIDDEN), jnp.float32),
        jax.ShapeDtypeStruct((BATCH, HIDDEN), jnp.float32),
    ],
    scratch_types=[
        pltpu.VMEM((_GCH, CHUNK), jnp.int32),
        pltpu.VMEM((_GCH, CHUNK), jnp.int32),
        pltpu.VMEM((2, CHUNK, HIDDEN), jnp.float32),
        pltpu.SemaphoreType.DMA,
        pltpu.SemaphoreType.DMA,
        pltpu.SemaphoreType.DMA,
        pltpu.SemaphoreType.DMA,
    ],
)
def _pair_gather(x_hbm, cid_hbm, did_hbm, cout_hbm, dout_hbm,
                 cidx, didx, rows, gs0, gs1, os0, os1):
    cid = lax.axis_index("c")
    sid = lax.axis_index("s")
    wid = sid * 2 + cid
    base = wid * B_TILE
    gsem = (gs0, gs1)
    osem = (os0, os1)

    pltpu.sync_copy(cid_hbm.at[wid], cidx)
    pltpu.sync_copy(did_hbm.at[wid], didx)

    def g(t, b):
        slot = (cidx if t < _GCH else didx).at[t % _GCH]
        return pltpu.make_async_copy(x_hbm.at[slot], rows.at[b], gsem[b])

    def o(t, b):
        out = cout_hbm if t < _GCH else dout_hbm
        return pltpu.make_async_copy(
            rows.at[b], out.at[pl.ds(base + (t % _GCH) * CHUNK, CHUNK)],
            osem[b])

    g(0, 0).start()
    for t in range(2 * _GCH):
        b = t % 2
        g(t, b).wait()
        o(t, b).start()
        if t + 1 < 2 * _GCH:
            if t >= 1:
                o(t - 1, 1 - b).wait()
            g(t + 1, 1 - b).start()
    o(2 * _GCH - 2, 0).wait()
    o(2 * _GCH - 1, 1).wait()


def _combine_body(x_ref, s_ref, cnt_ref, wroot_ref, wrel_ref, b_ref, o_ref,
                  *, relu):
    scale = 1.0 / jnp.maximum(cnt_ref[...], 1.0)
    acc = jnp.dot(x_ref[...], wroot_ref[...],
                  preferred_element_type=jnp.float32)
    acc = acc + b_ref[...]
    acc = acc + jnp.dot(s_ref[...] * scale, wrel_ref[0],
                        preferred_element_type=jnp.float32)
    if relu:
        acc = jnp.maximum(acc, 0.0)
    o_ref[...] = acc


_CBLK = 512
_NCB = NC_P // _CBLK


def _combine(x, s, cnt, wroot, wrel_l, b, relu):
    kern = functools.partial(_combine_body, relu=relu)
    return pl.pallas_call(
        kern,
        grid=(NN_P // _CBLK,),
        in_specs=[
            pl.BlockSpec((_CBLK, HIDDEN), lambda i: (i, 0)),
            pl.BlockSpec((_CBLK, HIDDEN), lambda i: (i, 0)),
            pl.BlockSpec((_CBLK, 1), lambda i: (i, 0)),
            pl.BlockSpec((HIDDEN, HIDDEN), lambda i: (0, 0)),
            pl.BlockSpec((1, HIDDEN, HIDDEN),
                         lambda i: (jnp.where(i < _NCB, 1, 0), 0, 0)),
            pl.BlockSpec((1, HIDDEN), lambda i: (0, 0)),
        ],
        out_specs=pl.BlockSpec((_CBLK, HIDDEN), lambda i: (i, 0)),
        out_shape=jax.ShapeDtypeStruct((NN_P, HIDDEN), jnp.float32),
    )(x, s, cnt, wroot, wrel_l, b)


def _score_body(c_ref, d_ref, w_ref, o_ref):
    cw = jnp.dot(c_ref[...], w_ref[...], preferred_element_type=jnp.float32)
    o_ref[...] = jnp.sum(cw * d_ref[...], axis=1, keepdims=True)


def _score(c, d, w):
    blk = 512
    return pl.pallas_call(
        _score_body,
        grid=(BATCH // blk,),
        in_specs=[
            pl.BlockSpec((blk, HIDDEN), lambda i: (i, 0)),
            pl.BlockSpec((blk, HIDDEN), lambda i: (i, 0)),
            pl.BlockSpec((HIDDEN, HIDDEN), lambda i: (0, 0)),
        ],
        out_specs=pl.BlockSpec((blk, 1), lambda i: (i, 0)),
        out_shape=jax.ShapeDtypeStruct((BATCH, 1), jnp.float32),
    )(c, d, w)


def kernel(chem_ids, dis_ids, train_chem, train_dis, node_emb, w_rel, w_root,
           bias, W):
    f32 = jnp.float32
    pad_e = E_PAD - E_POS
    tc_i = train_chem.astype(jnp.int32)
    td_i = train_dis.astype(jnp.int32)
    tc_pad = jnp.concatenate(
        [tc_i, jnp.full((pad_e,), NUM_CHEM, jnp.int32)]
    ).reshape(NT, NCHUNK, CHUNK)
    td_pad = jnp.concatenate(
        [td_i, jnp.full((pad_e,), NUM_DIS, jnp.int32)]
    ).reshape(NT, NCHUNK, CHUNK)
    td_ofs = td_pad + NC_P

    x = (jnp.zeros((NN_P, HIDDEN), f32)
         .at[:NUM_CHEM].set(node_emb[:NUM_CHEM])
         .at[NC_P:NC_P + NUM_DIS].set(node_emb[NUM_CHEM:]))

    cnt_keep = None
    for l in range(2):
        if l == 0:
            s, cnt = _seg_sums_l1(x, tc_pad, td_pad, td_ofs)
            cnt_keep = cnt.reshape(NN_P, 1)
        else:
            s = _seg_sums_l2(x, tc_pad, td_pad, td_ofs)
            if isinstance(s, (list, tuple)):
                s = s[0]
        x = _combine(x, s, cnt_keep, w_root[l], w_rel[l],
                     bias[l].reshape(1, HIDDEN), relu=(l == 0))

    cid3 = chem_ids.astype(jnp.int32).reshape(32, _GCH, CHUNK)
    did3 = (dis_ids.astype(jnp.int32) + NC_P).reshape(32, _GCH, CHUNK)
    c_rows, d_rows = _pair_gather(x, cid3, did3)
    return _score(c_rows, d_rows, W)[:, 0]


# speedup vs baseline: 1.0576x; 1.0022x over previous
import functools

import jax
import jax.numpy as jnp
from jax import lax
from jax.experimental import pallas as pl
from jax.experimental.pallas import tpu as pltpu
from jax.experimental.pallas import tpu_sc as plsc

NUM_CHEM = 6000
NUM_DIS = 4000
HIDDEN = 128
E_POS = 160000
BATCH = 16384

NC_P = 6144
ND_P = 4096
NN_P = NC_P + ND_P

NT = 16
CHUNK = 128
E_TILE = 10240
E_PAD = NT * E_TILE
NCHUNK = E_TILE // CHUNK

_mesh = plsc.VectorSubcoreMesh(core_axis_name="c", subcore_axis_name="s")


def _make_seg_sums(with_cnt):
    out_type = [
        jax.ShapeDtypeStruct((NN_P, HIDDEN), jnp.float32),
    ]
    scratch = [
        pltpu.VMEM((8, CHUNK), jnp.int32),
        pltpu.VMEM((8, CHUNK), jnp.int32),
        pltpu.VMEM((4, CHUNK, HIDDEN), jnp.float32),
        pltpu.VMEM((32, HIDDEN), jnp.float32),
        pltpu.VMEM((CHUNK,), jnp.float32),
        pltpu.VMEM_SHARED((NC_P, HIDDEN), jnp.float32),
        pltpu.VMEM_SHARED((NC_P,), jnp.float32),
    ] + [pltpu.SemaphoreType.DMA] * 17
    if with_cnt:
        out_type = out_type + [
            jax.ShapeDtypeStruct((NN_P,), jnp.float32),
        ]
    @functools.partial(pl.kernel, mesh=_mesh, out_type=out_type,
                       scratch_types=scratch)
    def _seg(x_hbm, tc_hbm, td_hbm, tdo_hbm, sum_hbm, *rest):
        if with_cnt:
            cnt_hbm = rest[0]
            rest = rest[1:]
        (sidx, didx, rows, zbuf, ones, acc, cacc, *sems) = rest
        gsem, ssem, isem, csem = sems[0:4], sems[4:8], sems[8:16], sems[16]
        cid = lax.axis_index("c")
        sid = lax.axis_index("s")

        for i in range(32):
            for j in range(HIDDEN // 16):
                zbuf[i, pl.ds(j * 16, 16)] = jnp.zeros((16,), jnp.float32)
        for j in range(CHUNK // 16):
            ones[pl.ds(j * 16, 16)] = jnp.ones((16,), jnp.float32)

        r0 = ND_P // NT
        r1 = NC_P // NT

        def zero_descs():
            ds = [pltpu.make_async_copy(
                      zbuf, acc.at[pl.ds(sid * r1 + b * 32, 32)],
                      sems[b % 4]) for b in range(r1 // 32)]
            if with_cnt:
                ds += [pltpu.make_async_copy(
                           zbuf.at[0],
                           cacc.at[pl.ds(sid * r1 + b * CHUNK, CHUNK)],
                           sems[4 + b]) for b in range(r1 // CHUNK)]
            return ds

        for d in zero_descs():
            d.start()
        for d in zero_descs():
            d.wait()
        plsc.subcore_barrier()

        def do_rel(src3_hbm, x_hbm, dst3_hbm, acc):
            def idx(k, i):
                return (pltpu.make_async_copy(
                            src3_hbm.at[sid].at[k], sidx.at[i], isem[i]),
                        pltpu.make_async_copy(
                            dst3_hbm.at[sid].at[k], didx.at[i], isem[i]))

            def gather(k, i, b):
                return pltpu.make_async_copy(
                    x_hbm.at[sidx.at[i]], rows.at[b], gsem[b])

            def scatter(k, i, b):
                return pltpu.make_async_copy(
                    rows.at[b], acc.at[didx.at[i]], ssem[b])

            for k in range(4):
                a, d = idx(k, k)
                a.start()
                d.start()
            for k in range(2):
                a, d = idx(k, k)
                a.wait()
                d.wait()
                gather(k, k, k).start()

            def body(j, carry):
                for b in range(8):
                    k = 8 * j + b
                    br = b % 4

                    @pl.when(k >= 2)
                    def _():
                        scatter(k - 2, (b + 6) % 8, (br + 2) % 4).wait()
                    gather(k, b, br).wait()
                    pltpu.async_copy(rows.at[br], acc.at[didx.at[b]],
                                     ssem[br], add=True)
                    if with_cnt:
                        pltpu.async_copy(ones, cacc.at[didx.at[b]], csem,
                                         add=True)

                    @pl.when(k + 2 < NCHUNK)
                    def _():
                        a, d = idx(k + 2, (b + 2) % 8)
                        a.wait()
                        d.wait()
                        gather(k + 2, (b + 2) % 8, (br + 2) % 4).start()

                    @pl.when(k + 4 < NCHUNK)
                    def _():
                        a, d = idx(k + 4, (b + 4) % 8)
                        a.start()
                        d.start()
                return carry
            lax.fori_loop(0, NCHUNK // 8, body, 0)

            scatter(NCHUNK - 2, 6, 2).wait()
            scatter(NCHUNK - 1, 7, 3).wait()
            if with_cnt:
                def cbody(k, carry):
                    pltpu.make_async_copy(
                        ones, cacc.at[didx.at[0]], csem).wait()
                    return carry
                lax.fori_loop(0, NCHUNK, cbody, 0)

        @pl.when(cid == 0)
        def _():
            do_rel(tc_hbm, x_hbm, td_hbm, acc)

        @pl.when(cid == 1)
        def _():
            do_rel(tdo_hbm, x_hbm, tc_hbm, acc)

        plsc.subcore_barrier()

        @pl.when(cid == 0)
        def _():
            pltpu.sync_copy(acc.at[pl.ds(sid * r0, r0)],
                            sum_hbm.at[pl.ds(NC_P + sid * r0, r0)])
            if with_cnt:
                @pl.when(sid == 0)
                def _():
                    pltpu.sync_copy(cacc.at[pl.ds(0, ND_P)],
                                    cnt_hbm.at[pl.ds(NC_P, ND_P)])

        @pl.when(cid == 1)
        def _():
            pltpu.sync_copy(acc.at[pl.ds(sid * r1, r1)],
                            sum_hbm.at[pl.ds(sid * r1, r1)])
            if with_cnt:
                @pl.when(sid == 0)
                def _():
                    pltpu.sync_copy(cacc, cnt_hbm.at[pl.ds(0, NC_P)])

    return _seg


_seg_sums_l1 = _make_seg_sums(True)
_seg_sums_l2 = _make_seg_sums(False)


B_TILE = BATCH // 32


_GCH = B_TILE // CHUNK


@functools.partial(
    pl.kernel,
    mesh=_mesh,
    out_type=[
        jax.ShapeDtypeStruct((BATCH, HIDDEN), jnp.float32),
        jax.ShapeDtypeStruct((BATCH, HIDDEN), jnp.float32),
    ],
    scratch_types=[
        pltpu.VMEM((_GCH, CHUNK), jnp.int32),
        pltpu.VMEM((_GCH, CHUNK), jnp.int32),
        pltpu.VMEM((2, CHUNK, HIDDEN), jnp.float32),
        pltpu.SemaphoreType.DMA,
        pltpu.SemaphoreType.DMA,
        pltpu.SemaphoreType.DMA,
        pltpu.SemaphoreType.DMA,
    ],
)
def _pair_gather(x_hbm, cid_hbm, did_hbm, cout_hbm, dout_hbm,
                 cidx, didx, rows, gs0, gs1, os0, os1):
    cid = lax.axis_index("c")
    sid = lax.axis_index("s")
    wid = sid * 2 + cid
    base = wid * B_TILE
    gsem = (gs0, gs1)
    osem = (os0, os1)

    pltpu.sync_copy(cid_hbm.at[wid], cidx)
    pltpu.sync_copy(did_hbm.at[wid], didx)

    def g(t, b):
        slot = (cidx if t < _GCH else didx).at[t % _GCH]
        return pltpu.make_async_copy(x_hbm.at[slot], rows.at[b], gsem[b])

    def o(t, b):
        out = cout_hbm if t < _GCH else dout_hbm
        return pltpu.make_async_copy(
            rows.at[b], out.at[pl.ds(base + (t % _GCH) * CHUNK, CHUNK)],
            osem[b])

    g(0, 0).start()
    for t in range(2 * _GCH):
        b = t % 2
        g(t, b).wait()
        o(t, b).start()
        if t + 1 < 2 * _GCH:
            if t >= 1:
                o(t - 1, 1 - b).wait()
            g(t + 1, 1 - b).start()
    o(2 * _GCH - 2, 0).wait()
    o(2 * _GCH - 1, 1).wait()


def _combine_body(x_ref, s_ref, cnt_ref, wroot_ref, wrel_ref, b_ref, o_ref,
                  *, relu):
    scale = 1.0 / jnp.maximum(cnt_ref[...], 1.0)
    acc = jnp.dot(x_ref[...], wroot_ref[...],
                  preferred_element_type=jnp.float32)
    acc = acc + b_ref[...]
    acc = acc + jnp.dot(s_ref[...] * scale, wrel_ref[0],
                        preferred_element_type=jnp.float32)
    if relu:
        acc = jnp.maximum(acc, 0.0)
    o_ref[...] = acc


_CBLK = 512
_NCB = NC_P // _CBLK


def _combine(x, s, cnt, wroot, wrel_l, b, relu):
    kern = functools.partial(_combine_body, relu=relu)
    return pl.pallas_call(
        kern,
        grid=(NN_P // _CBLK,),
        in_specs=[
            pl.BlockSpec((_CBLK, HIDDEN), lambda i: (i, 0)),
            pl.BlockSpec((_CBLK, HIDDEN), lambda i: (i, 0)),
            pl.BlockSpec((_CBLK, 1), lambda i: (i, 0)),
            pl.BlockSpec((HIDDEN, HIDDEN), lambda i: (0, 0)),
            pl.BlockSpec((1, HIDDEN, HIDDEN),
                         lambda i: (jnp.where(i < _NCB, 1, 0), 0, 0)),
            pl.BlockSpec((1, HIDDEN), lambda i: (0, 0)),
        ],
        out_specs=pl.BlockSpec((_CBLK, HIDDEN), lambda i: (i, 0)),
        out_shape=jax.ShapeDtypeStruct((NN_P, HIDDEN), jnp.float32),
    )(x, s, cnt, wroot, wrel_l, b)


def _score_body(c_ref, d_ref, w_ref, o_ref):
    cw = jnp.dot(c_ref[...], w_ref[...], preferred_element_type=jnp.float32)
    o_ref[...] = jnp.sum(cw * d_ref[...], axis=1, keepdims=True)


def _score(c, d, w):
    blk = 512
    return pl.pallas_call(
        _score_body,
        grid=(BATCH // blk,),
        in_specs=[
            pl.BlockSpec((blk, HIDDEN), lambda i: (i, 0)),
            pl.BlockSpec((blk, HIDDEN), lambda i: (i, 0)),
            pl.BlockSpec((HIDDEN, HIDDEN), lambda i: (0, 0)),
        ],
        out_specs=pl.BlockSpec((blk, 1), lambda i: (i, 0)),
        out_shape=jax.ShapeDtypeStruct((BATCH, 1), jnp.float32),
    )(c, d, w)


def kernel(chem_ids, dis_ids, train_chem, train_dis, node_emb, w_rel, w_root,
           bias, W):
    f32 = jnp.float32
    pad_e = E_PAD - E_POS
    tc_i = train_chem.astype(jnp.int32)
    td_i = train_dis.astype(jnp.int32)
    tc_pad = jnp.concatenate(
        [tc_i, jnp.full((pad_e,), NUM_CHEM, jnp.int32)]
    ).reshape(NT, NCHUNK, CHUNK)
    td_pad = jnp.concatenate(
        [td_i, jnp.full((pad_e,), NUM_DIS, jnp.int32)]
    ).reshape(NT, NCHUNK, CHUNK)
    td_ofs = td_pad + NC_P

    x = (jnp.zeros((NN_P, HIDDEN), f32)
         .at[:NUM_CHEM].set(node_emb[:NUM_CHEM])
         .at[NC_P:NC_P + NUM_DIS].set(node_emb[NUM_CHEM:]))

    cnt_keep = None
    for l in range(2):
        if l == 0:
            s, cnt = _seg_sums_l1(x, tc_pad, td_pad, td_ofs)
            cnt_keep = cnt.reshape(NN_P, 1)
        else:
            s = _seg_sums_l2(x, tc_pad, td_pad, td_ofs)
            if isinstance(s, (list, tuple)):
                s = s[0]
        x = _combine(x, s, cnt_keep, w_root[l], w_rel[l],
                     bias[l].reshape(1, HIDDEN), relu=(l == 0))

    cid3 = chem_ids.astype(jnp.int32).reshape(32, _GCH, CHUNK)
    did3 = (dis_ids.astype(jnp.int32) + NC_P).reshape(32, _GCH, CHUNK)
    c_rows, d_rows = _pair_gather(x, cid3, did3)
    return _score(c_rows, d_rows, W)[:, 0]
